# Initial kernel scaffold; baseline (speedup 1.0000x reference)
import jax
import jax.numpy as jnp
from jax.experimental import pallas as pl


def kernel(position_ids, weight):
    raise NotImplementedError("write your pallas kernel here")


# speedup vs baseline: 1.6284x; 1.6284x over previous
import functools

import jax
import jax.numpy as jnp
from jax import lax
from jax.experimental import pallas as pl
from jax.experimental.pallas import tpu as pltpu
from jax.experimental.pallas import tpu_sc as plsc

_NUM_CORES = 2
_NUM_SUBCORES = 16
_NUM_WORKERS = _NUM_CORES * _NUM_SUBCORES
_CHUNK = 32


def _gather_call(total, hidden, idx, table):
    b_per_w = total // _NUM_WORKERS
    n_chunks = b_per_w // _CHUNK
    mesh = plsc.VectorSubcoreMesh(core_axis_name="c", subcore_axis_name="s")

    @functools.partial(
        pl.kernel,
        mesh=mesh,
        out_type=jax.ShapeDtypeStruct((total, hidden), jnp.float32),
        scratch_types=[
            pltpu.VMEM((b_per_w,), jnp.int32),
            pltpu.VMEM((_CHUNK, hidden), jnp.float32),
            pltpu.SemaphoreType.DMA,
        ],
    )
    def _gather(idx_hbm, table_hbm, out_hbm, idx_v, rows_v, sem):
        wid = lax.axis_index("s") * _NUM_CORES + lax.axis_index("c")
        base = wid * b_per_w
        pltpu.sync_copy(idx_hbm.at[pl.ds(base, b_per_w)], idx_v)

        def body(c, carry):
            start = c * _CHUNK
            pltpu.async_copy(
                table_hbm.at[idx_v.at[pl.ds(start, _CHUNK)]], rows_v, sem
            ).wait()
            pltpu.sync_copy(rows_v, out_hbm.at[pl.ds(base + start, _CHUNK)])
            return carry

        lax.fori_loop(0, n_chunks, body, 0)

    return _gather(idx, table)


def kernel(position_ids, weight):
    batch, seq = position_ids.shape
    vocab, hidden = weight.shape
    total = batch * seq
    idx = position_ids.reshape(total).astype(jnp.int32)
    out = _gather_call(total, hidden, idx, weight)
    return out.reshape(batch, seq, hidden)


# speedup vs baseline: 1.7882x; 1.0982x over previous
import functools

import jax
import jax.numpy as jnp
from jax import lax
from jax.experimental import pallas as pl
from jax.experimental.pallas import tpu as pltpu
from jax.experimental.pallas import tpu_sc as plsc

_NUM_CORES = 2
_NUM_SUBCORES = 16
_NUM_WORKERS = _NUM_CORES * _NUM_SUBCORES
_CHUNK = 16


def _gather_call(total, hidden, idx, table):
    b_per_w = total // _NUM_WORKERS
    n_chunks = b_per_w // _CHUNK
    mesh = plsc.VectorSubcoreMesh(core_axis_name="c", subcore_axis_name="s")

    @functools.partial(
        pl.kernel,
        mesh=mesh,
        out_type=jax.ShapeDtypeStruct((total, hidden), jnp.float32),
        scratch_types=[
            pltpu.VMEM((b_per_w,), jnp.int32),
            pltpu.VMEM((_CHUNK, hidden), jnp.float32),
            pltpu.VMEM((_CHUNK, hidden), jnp.float32),
            pltpu.SemaphoreType.DMA,
            pltpu.SemaphoreType.DMA,
        ],
    )
    def _gather(idx_hbm, table_hbm, out_hbm, idx_v, buf0, buf1, sem_in, sem_out):
        wid = lax.axis_index("s") * _NUM_CORES + lax.axis_index("c")
        base = wid * b_per_w
        pltpu.sync_copy(idx_hbm.at[pl.ds(base, b_per_w)], idx_v)

        def gather_start(c, buf):
            pltpu.async_copy(
                table_hbm.at[idx_v.at[pl.ds(c * _CHUNK, _CHUNK)]], buf, sem_in
            )

        def gather_wait(buf):
            pltpu.make_async_copy(
                table_hbm.at[pl.ds(0, _CHUNK)], buf, sem_in
            ).wait()

        def scatter_start(c, buf):
            pltpu.async_copy(buf, out_hbm.at[pl.ds(base + c * _CHUNK, _CHUNK)], sem_out)

        def scatter_wait(c, buf):
            pltpu.make_async_copy(
                buf, out_hbm.at[pl.ds(base + c * _CHUNK, _CHUNK)], sem_out
            ).wait()

        gather_start(0, buf0)
        gather_start(1, buf1)

        def body(c2, carry):
            c = c2 * 2
            for j, buf in ((0, buf0), (1, buf1)):
                gather_wait(buf)
                scatter_start(c + j, buf)
                scatter_wait(c + j, buf)
                gather_start(c + j + 2, buf)
            return carry

        lax.fori_loop(0, n_chunks // 2 - 1, body, 0)

        c = n_chunks - 2
        for j, buf in ((0, buf0), (1, buf1)):
            gather_wait(buf)
            scatter_start(c + j, buf)
        for j, buf in ((0, buf0), (1, buf1)):
            scatter_wait(c + j, buf)

    return _gather(idx, table)


def kernel(position_ids, weight):
    batch, seq = position_ids.shape
    vocab, hidden = weight.shape
    total = batch * seq
    idx = position_ids.reshape(total).astype(jnp.int32)
    out = _gather_call(total, hidden, idx, weight)
    return out.reshape(batch, seq, hidden)
